# baseline (speedup 1.0000x reference)
import functools

import jax
import jax.numpy as jnp
from jax import lax
from jax.experimental import pallas as pl
from jax.experimental.pallas import tpu as pltpu
from jax.experimental.pallas import tpu_sc as plsc

INPUT_SIZE = 512
CONTEXT_SIZE = 256
CONTEXT_MAP_SIZE = 4
BATCH = 4096
NUM_CTX = 2 ** CONTEXT_MAP_SIZE

SC_CORES = 2
SC_SUBCORES = 16
SC_LANES = 16
NUM_WORKERS = SC_CORES * SC_SUBCORES
BPW = BATCH // NUM_WORKERS
CROWS = 24


def _tc_body(x_ref, c_ref, p_ref, b_ref, w_ref, v_ref, cmb_ref):
    pj = lax.dot_general(
        p_ref[...], c_ref[...], (((1,), (0,)), ((), ())),
        preferred_element_type=jnp.float32)
    bits = pj > b_ref[...]
    idx_row = jnp.sum(jnp.where(bits, v_ref[...], 0.0), axis=0,
                      keepdims=True)
    cmb_ref[0, 0:1, :] = idx_row
    cmb_ref[0, 1:1 + NUM_CTX, :] = lax.dot_general(
        w_ref[...], x_ref[...], (((1,), (0,)), ((), ())),
        preferred_element_type=jnp.float32)
    cmb_ref[0, 1 + NUM_CTX:, :] = jnp.zeros(
        (CROWS - 1 - NUM_CTX, BPW), jnp.float32)


def _sc_gather(cmb_hbm, out_hbm, tab_v, out_v):
    wid = lax.axis_index("s") * SC_CORES + lax.axis_index("c")
    base = wid * BPW
    pltpu.sync_copy(cmb_hbm.at[wid], tab_v)
    for i in range(BPW // SC_LANES):
        rows = tab_v[0, pl.ds(i * SC_LANES, SC_LANES)].astype(jnp.int32)
        b_loc = lax.iota(jnp.int32, SC_LANES) + (i * SC_LANES)
        out_v[pl.ds(i * SC_LANES, SC_LANES)] = plsc.load_gather(
            tab_v, [rows + 1, b_loc])
    pltpu.sync_copy(out_v, out_hbm.at[pl.ds(base, BPW)])


def kernel(logits, context_inputs, projection, projection_bias, weights,
           boolean_converter):
    f32 = jnp.float32

    cmb = pl.pallas_call(
        _tc_body,
        grid=(NUM_WORKERS,),
        in_specs=[
            pl.BlockSpec((INPUT_SIZE, BPW), lambda i: (0, i)),
            pl.BlockSpec((CONTEXT_SIZE, BPW), lambda i: (0, i)),
            pl.BlockSpec((CONTEXT_MAP_SIZE, CONTEXT_SIZE), lambda i: (0, 0)),
            pl.BlockSpec((CONTEXT_MAP_SIZE, 1), lambda i: (0, 0)),
            pl.BlockSpec((NUM_CTX, INPUT_SIZE), lambda i: (0, 0)),
            pl.BlockSpec((CONTEXT_MAP_SIZE, 1), lambda i: (0, 0)),
        ],
        out_specs=[
            pl.BlockSpec((1, CROWS, BPW), lambda i: (i, 0, 0)),
        ],
        out_shape=[
            jax.ShapeDtypeStruct((NUM_WORKERS, CROWS, BPW), f32),
        ],
    )(logits, context_inputs, projection, projection_bias, weights,
      boolean_converter)[0]

    sc_fn = functools.partial(
        pl.kernel,
        mesh=plsc.VectorSubcoreMesh(core_axis_name="c", subcore_axis_name="s"),
        out_type=jax.ShapeDtypeStruct((BATCH,), f32),
        scratch_types=[
            pltpu.VMEM((CROWS, BPW), f32),
            pltpu.VMEM((BPW,), f32),
        ],
        compiler_params=pltpu.CompilerParams(needs_layout_passes=False),
    )(_sc_gather)
    return sc_fn(cmb)


# speedup vs baseline: 1.5755x; 1.5755x over previous
import functools

import jax
import jax.numpy as jnp
from jax import lax
from jax.experimental import pallas as pl
from jax.experimental.pallas import tpu as pltpu
from jax.experimental.pallas import tpu_sc as plsc

INPUT_SIZE = 512
CONTEXT_SIZE = 256
CONTEXT_MAP_SIZE = 4
BATCH = 4096
NUM_CTX = 2 ** CONTEXT_MAP_SIZE

SC_CORES = 2
SC_SUBCORES = 16
SC_LANES = 16
NUM_WORKERS = SC_CORES * SC_SUBCORES
BPW = BATCH // NUM_WORKERS
CROWS = 24


_BC = 2048
_CPG = _BC // BPW


def _tc_body(x_ref, c_ref, p_ref, b_ref, w_ref, v_ref, cmb_ref):
    pj = lax.dot_general(
        p_ref[...], c_ref[...], (((1,), (0,)), ((), ())),
        preferred_element_type=jnp.float32)
    bits = pj > b_ref[...]
    idx_row = jnp.sum(jnp.where(bits, v_ref[...], 0.0), axis=0,
                      keepdims=True)
    a16 = lax.dot_general(
        w_ref[...], x_ref[...], (((1,), (0,)), ((), ())),
        preferred_element_type=jnp.float32)
    for t in range(_CPG):
        lo, hi = t * BPW, (t + 1) * BPW
        cmb_ref[t, 0:1, :] = idx_row[:, lo:hi]
        cmb_ref[t, 1:1 + NUM_CTX, :] = a16[:, lo:hi]


def _sc_gather(cmb_hbm, out_hbm, tab_v, out_v):
    wid = lax.axis_index("s") * SC_CORES + lax.axis_index("c")
    base = wid * BPW
    pltpu.sync_copy(cmb_hbm.at[wid], tab_v)
    for i in range(BPW // SC_LANES):
        rows = tab_v[0, pl.ds(i * SC_LANES, SC_LANES)].astype(jnp.int32)
        b_loc = lax.iota(jnp.int32, SC_LANES) + (i * SC_LANES)
        out_v[pl.ds(i * SC_LANES, SC_LANES)] = plsc.load_gather(
            tab_v, [rows + 1, b_loc])
    pltpu.sync_copy(out_v, out_hbm.at[pl.ds(base, BPW)])


def kernel(logits, context_inputs, projection, projection_bias, weights,
           boolean_converter):
    f32 = jnp.float32

    cmb = pl.pallas_call(
        _tc_body,
        grid=(BATCH // _BC,),
        in_specs=[
            pl.BlockSpec((INPUT_SIZE, _BC), lambda i: (0, i)),
            pl.BlockSpec((CONTEXT_SIZE, _BC), lambda i: (0, i)),
            pl.BlockSpec((CONTEXT_MAP_SIZE, CONTEXT_SIZE), lambda i: (0, 0)),
            pl.BlockSpec((CONTEXT_MAP_SIZE, 1), lambda i: (0, 0)),
            pl.BlockSpec((NUM_CTX, INPUT_SIZE), lambda i: (0, 0)),
            pl.BlockSpec((CONTEXT_MAP_SIZE, 1), lambda i: (0, 0)),
        ],
        out_specs=[
            pl.BlockSpec((_CPG, CROWS, BPW), lambda i: (i, 0, 0)),
        ],
        out_shape=[
            jax.ShapeDtypeStruct((NUM_WORKERS, CROWS, BPW), f32),
        ],
    )(logits, context_inputs, projection, projection_bias, weights,
      boolean_converter)[0]

    sc_fn = functools.partial(
        pl.kernel,
        mesh=plsc.VectorSubcoreMesh(core_axis_name="c", subcore_axis_name="s"),
        out_type=jax.ShapeDtypeStruct((BATCH,), f32),
        scratch_types=[
            pltpu.VMEM((CROWS, BPW), f32),
            pltpu.VMEM((BPW,), f32),
        ],
        compiler_params=pltpu.CompilerParams(needs_layout_passes=False),
    )(_sc_gather)
    return sc_fn(cmb)


# speedup vs baseline: 1.7549x; 1.1139x over previous
import functools

import jax
import jax.numpy as jnp
from jax import lax
from jax.experimental import pallas as pl
from jax.experimental.pallas import tpu as pltpu
from jax.experimental.pallas import tpu_sc as plsc

INPUT_SIZE = 512
CONTEXT_SIZE = 256
CONTEXT_MAP_SIZE = 4
BATCH = 4096
NUM_CTX = 2 ** CONTEXT_MAP_SIZE

SC_CORES = 2
SC_SUBCORES = 16
SC_LANES = 16
NUM_WORKERS = SC_CORES * SC_SUBCORES
BPW = BATCH // NUM_WORKERS
CROWS = 24


_BC = 2048
_CPG = _BC // BPW


def _tc_body(x_ref, c_ref, p_ref, b_ref, w_ref, v_ref, cmb_ref):
    pj = lax.dot_general(
        p_ref[...], c_ref[...], (((1,), (0,)), ((), ())),
        preferred_element_type=jnp.float32)
    idx_row = jnp.zeros((1, _BC), jnp.float32)
    for j in range(CONTEXT_MAP_SIZE):
        idx_row = idx_row + jnp.where(pj[j:j + 1, :] > b_ref[j], v_ref[j],
                                      0.0)
    a16 = lax.dot_general(
        w_ref[...], x_ref[...], (((1,), (0,)), ((), ())),
        preferred_element_type=jnp.float32)
    for t in range(_CPG):
        lo, hi = t * BPW, (t + 1) * BPW
        cmb_ref[t, 0:1, :] = idx_row[:, lo:hi]
        cmb_ref[t, 1:1 + NUM_CTX, :] = a16[:, lo:hi]


def _sc_gather(cmb_hbm, out_hbm, tab_v, out_v):
    wid = lax.axis_index("s") * SC_CORES + lax.axis_index("c")
    base = wid * BPW
    pltpu.sync_copy(cmb_hbm.at[wid], tab_v)
    for i in range(BPW // SC_LANES):
        rows = tab_v[0, pl.ds(i * SC_LANES, SC_LANES)].astype(jnp.int32)
        b_loc = lax.iota(jnp.int32, SC_LANES) + (i * SC_LANES)
        out_v[pl.ds(i * SC_LANES, SC_LANES)] = plsc.load_gather(
            tab_v, [rows + 1, b_loc])
    pltpu.sync_copy(out_v, out_hbm.at[pl.ds(base, BPW)])


def kernel(logits, context_inputs, projection, projection_bias, weights,
           boolean_converter):
    f32 = jnp.float32

    cmb = pl.pallas_call(
        _tc_body,
        grid=(BATCH // _BC,),
        in_specs=[
            pl.BlockSpec((INPUT_SIZE, _BC), lambda i: (0, i)),
            pl.BlockSpec((CONTEXT_SIZE, _BC), lambda i: (0, i)),
            pl.BlockSpec((CONTEXT_MAP_SIZE, CONTEXT_SIZE), lambda i: (0, 0)),
            pl.BlockSpec(memory_space=pltpu.SMEM),
            pl.BlockSpec((NUM_CTX, INPUT_SIZE), lambda i: (0, 0)),
            pl.BlockSpec(memory_space=pltpu.SMEM),
        ],
        out_specs=[
            pl.BlockSpec((_CPG, CROWS, BPW), lambda i: (i, 0, 0)),
        ],
        out_shape=[
            jax.ShapeDtypeStruct((NUM_WORKERS, CROWS, BPW), f32),
        ],
    )(logits, context_inputs, projection, projection_bias.reshape(-1),
      weights, boolean_converter.reshape(-1))[0]

    sc_fn = functools.partial(
        pl.kernel,
        mesh=plsc.VectorSubcoreMesh(core_axis_name="c", subcore_axis_name="s"),
        out_type=jax.ShapeDtypeStruct((BATCH,), f32),
        scratch_types=[
            pltpu.VMEM((CROWS, BPW), f32),
            pltpu.VMEM((BPW,), f32),
        ],
        compiler_params=pltpu.CompilerParams(needs_layout_passes=False),
    )(_sc_gather)
    return sc_fn(cmb)
